# baseline (speedup 1.0000x reference)
import functools

import jax
import jax.numpy as jnp
from jax import lax
from jax.experimental import pallas as pl
from jax.experimental.pallas import tpu as pltpu
from jax.experimental.pallas import tpu_sc as plsc

_DP = 128


def _make_sc_gather(V, D, B, NC, NS):
    NW = NC * NS
    NSLAB = (V + 127) // 128
    G = 6
    NV = B // 16
    CAP = B + 16
    mesh = plsc.VectorSubcoreMesh(core_axis_name="c", subcore_axis_name="s")

    @functools.partial(
        pl.kernel,
        mesh=mesh,
        out_type=jax.ShapeDtypeStruct((B + 128, _DP), jnp.float32),
        scratch_types=[
            pltpu.VMEM((B,), jnp.int32),
            pltpu.VMEM((CAP,), jnp.int32),
            pltpu.VMEM((CAP,), jnp.int32),
            pltpu.VMEM((G, D, 128), jnp.float32),
            pltpu.VMEM((128, _DP), jnp.float32),
            pltpu.VMEM((1, 128), jnp.int32),
            pltpu.VMEM((32,), jnp.int32),
            pltpu.VMEM((32,), jnp.int32),
            pltpu.SemaphoreType.DMA((G,)),
            pltpu.SemaphoreType.DMA,
        ],
        compiler_params=pltpu.CompilerParams(
            use_tc_tiling_on_sc=True, needs_layout_passes=False),
    )
    def gk(tab, idx_hbm, out, idx_v, lid, lpos, bufs, stg, pstage, scr_i,
           scr_p, sems, sem_sc):
        wid = lax.axis_index("s") * NC + lax.axis_index("c")
        lo = (NSLAB * wid) // NW
        hi = (NSLAB * (wid + 1)) // NW
        iota = lax.broadcasted_iota(jnp.int32, (16,), 0)
        zeros16 = jnp.zeros((16,), jnp.float32)

        pltpu.sync_copy(idx_hbm, idx_v)

        def scan_body(k, cnt):
            ids = idx_v[pl.ds(16 * k, 16)]
            slab = lax.shift_right_logical(ids, 7)
            m = (slab >= lo) & (slab < hi)
            plsc.store_compressed(lid.at[pl.ds(cnt, 16)], ids, mask=m)
            plsc.store_compressed(lpos.at[pl.ds(cnt, 16)], iota + 16 * k, mask=m)
            return cnt + plsc.all_reduce_population_count(m)[0]

        cnt = lax.fori_loop(0, NV, scan_body, 0, unroll=2)
        nvl = (cnt + 15) >> 4

        def zrow(r, c):
            for c8 in range(_DP // 16):
                stg[r, pl.ds(16 * c8, 16)] = zeros16
            return c

        lax.fori_loop(0, 128, zrow, 0)

        def fill_dummies():
            for k in range(8):
                pstage[0, pl.ds(16 * k, 16)] = B + iota + 16 * k

        fill_dummies()

        def fetch(b, s):
            pltpu.async_copy(tab.at[:, pl.ds(s * 128, 128)],
                             bufs.at[b], sems.at[b])

        def flush():
            pltpu.async_copy(stg, out.at[pstage.at[0]], sem_sc).wait()
            fill_dummies()

        def group(gi, cursor):
            s0 = lo + gi * G

            for b in range(G):
                @pl.when(s0 + b < hi)
                def _(b=b):
                    fetch(b, s0 + b)

            def slab_work(b, cursor):
                s = s0 + b

                @pl.when(s < hi)
                def _():
                    pltpu.make_async_copy(tab.at[:, pl.ds(0, 128)],
                                          bufs.at[b], sems.at[b]).wait()

                def vreg_work(k2, cursor):
                    ids16 = lid[pl.ds(16 * k2, 16)]
                    pos16 = lpos[pl.ds(16 * k2, 16)]
                    in_range = iota < (cnt - 16 * k2)
                    m0 = (lax.shift_right_logical(ids16, 7) == s) & in_range
                    scr_i[pl.ds(0, 16)] = ids16
                    scr_p[pl.ds(0, 16)] = pos16
                    c16 = plsc.all_reduce_population_count(m0)[0]

                    def match_body(i, carry):
                        cursor, m = carry
                        lane = plsc.all_reduce_ffs(m)[0]
                        g = scr_i[pl.ds(lane, 16)][0]
                        p = scr_p[pl.ds(lane, 16)][0]
                        lvec = jnp.full((16,), 1, jnp.int32) * (g & 127)
                        row = cursor & 127
                        for c in range(4):
                            d_vec = jnp.minimum(iota + 16 * c, D - 1)
                            vals = plsc.load_gather(bufs.at[b], [d_vec, lvec])
                            stg[row, pl.ds(16 * c, 16)] = vals
                        pstage[0, pl.ds(row - (row & 15), 16)] = jnp.where(
                            iota == (row & 15), p,
                            pstage[0, pl.ds(row - (row & 15), 16)])

                        @pl.when(row == 127)
                        def _():
                            flush()

                        return cursor + 1, m & (iota != lane)

                    cursor, _ = lax.fori_loop(0, c16, match_body, (cursor, m0))
                    return cursor

                return lax.fori_loop(0, nvl, vreg_work, cursor)

            for b in range(G):
                cursor = slab_work(b, cursor)
            return cursor

        ngrp = (hi - lo + G - 1) // G
        lax.fori_loop(0, ngrp, group, 0)
        flush()

    return gk


def _make_tc_proj(B, DP, P, BB):
    grid = B // BB

    def proj_kernel(u_ref, id_ref, ww_ref, bw_ref, wa_ref, ba_ref, ow_ref, oa_ref):
        mask = (id_ref[...] != 0).astype(jnp.float32)
        x = u_ref[...] * mask
        yw = jnp.dot(x, ww_ref[...], preferred_element_type=jnp.float32) + bw_ref[...]
        ya = jnp.dot(x, wa_ref[...], preferred_element_type=jnp.float32) + ba_ref[...]
        ow_ref[...] = jnp.maximum(yw, 0.0)
        oa_ref[...] = jnp.maximum(ya, 0.0)

    return pl.pallas_call(
        proj_kernel,
        grid=(grid,),
        in_specs=[
            pl.BlockSpec((BB, DP), lambda i: (i, 0)),
            pl.BlockSpec((BB, 1), lambda i: (i, 0)),
            pl.BlockSpec((DP, P), lambda i: (0, 0)),
            pl.BlockSpec((1, P), lambda i: (0, 0)),
            pl.BlockSpec((DP, P), lambda i: (0, 0)),
            pl.BlockSpec((1, P), lambda i: (0, 0)),
        ],
        out_specs=[
            pl.BlockSpec((BB, P), lambda i: (i, 0)),
            pl.BlockSpec((BB, P), lambda i: (i, 0)),
        ],
        out_shape=[
            jax.ShapeDtypeStruct((B, P), jnp.float32),
            jax.ShapeDtypeStruct((B, P), jnp.float32),
        ],
    )


def kernel(user_id, table, W_word, b_word, W_article, b_article):
    B = user_id.shape[0]
    V, D = table.shape
    P = W_word.shape[1]
    info = plsc.get_sparse_core_info()
    NC, NS = info.num_cores, info.num_subcores

    idx = user_id.astype(jnp.int32)
    u = _make_sc_gather(V, D, B, NC, NS)(table.T, idx)
    Wwp = jnp.pad(W_word, ((0, _DP - D), (0, 0)))
    Wap = jnp.pad(W_article, ((0, _DP - D), (0, 0)))
    proj = _make_tc_proj(B, _DP, P, BB=2048)
    return tuple(
        proj(u, idx.reshape(B, 1), Wwp, b_word.reshape(1, P),
             Wap, b_article.reshape(1, P))
    )


# speedup vs baseline: 1.4123x; 1.4123x over previous
import functools

import jax
import jax.numpy as jnp
from jax import lax
from jax.experimental import pallas as pl
from jax.experimental.pallas import tpu as pltpu
from jax.experimental.pallas import tpu_sc as plsc

_DP = 128


def _make_sc_gather(V, D, B, NC, NS):
    NW = NC * NS
    NSLAB = (V + 127) // 128
    G = 4
    NBANK = 2
    NV = B // 16
    CAP = B + 32
    SPAN = NSLAB // NW + 1
    NGRP = (SPAN + G - 1) // G
    FLUSH = 64
    mesh = plsc.VectorSubcoreMesh(core_axis_name="c", subcore_axis_name="s")

    @functools.partial(
        pl.kernel,
        mesh=mesh,
        out_type=jax.ShapeDtypeStruct((B + 128, _DP), jnp.float32),
        scratch_types=[
            pltpu.VMEM((B,), jnp.int32),
            pltpu.VMEM((CAP,), jnp.int32),
            pltpu.VMEM((NBANK * G, D, 128), jnp.float32),
            pltpu.VMEM((FLUSH, _DP), jnp.float32),
            pltpu.VMEM((1, FLUSH), jnp.int32),
            pltpu.VMEM((272,), jnp.int32),
            pltpu.VMEM((32,), jnp.int32),
            pltpu.SemaphoreType.DMA((NBANK * G,)),
            pltpu.SemaphoreType.DMA,
        ],
        compiler_params=pltpu.CompilerParams(
            use_tc_tiling_on_sc=True, needs_layout_passes=False),
    )
    def gk(tab, idx_hbm, out, idx_v, lkey, bufs, stg, pstage, occ, scr,
           sems, sem_sc):
        wid = lax.axis_index("s") * NC + lax.axis_index("c")
        lo = (NSLAB * wid) // NW
        hi = (NSLAB * (wid + 1)) // NW
        iota = lax.broadcasted_iota(jnp.int32, (16,), 0)
        zeros16 = jnp.zeros((16,), jnp.float32)
        ones16 = jnp.full((16,), 1, jnp.int32)

        pltpu.sync_copy(idx_hbm, idx_v)

        for k in range(272 // 16):
            occ[pl.ds(16 * k, 16)] = iota * 0

        def scan_body(k, cnt):
            ids = idx_v[pl.ds(16 * k, 16)]
            rel = lax.shift_right_logical(ids, 7) - lo
            m = (rel >= 0) & (rel < hi - lo)
            relc = jnp.clip(rel, 0, 255)
            key = (relc << 21) | ((ids & 127) << 14) | (iota + 16 * k)
            plsc.store_compressed(lkey.at[pl.ds(cnt, 16)], key, mask=m)
            plsc.store_scatter(occ, [relc], ones16, mask=m)
            return cnt + plsc.all_reduce_population_count(m)[0]

        cnt = lax.fori_loop(0, NV, scan_body, 0)
        lkey[pl.ds(cnt, 16)] = iota * 0 - 1
        nvl = (cnt + 15) >> 4

        def zrow(r, c):
            for c8 in range(_DP // 16):
                stg[r, pl.ds(16 * c8, 16)] = zeros16
            return c

        lax.fori_loop(0, FLUSH, zrow, 0)

        def fill_dummies():
            for k in range(FLUSH // 16):
                pstage[0, pl.ds(16 * k, 16)] = B + iota + 16 * k

        fill_dummies()

        def occ_at(rel):
            return occ[pl.ds(jnp.clip(rel, 0, 255), 16)][0]

        def fire_group(gi, bank):
            for b in range(G):
                rel = gi * G + b
                s = lo + rel

                @pl.when((s < hi) & (occ_at(rel) > 0))
                def _(b=b, s=s, rel=rel):
                    pltpu.async_copy(tab.at[:, pl.ds(s * 128, 128)],
                                     bufs.at[bank * G + b], sems.at[bank * G + b])

        def flush():
            pltpu.async_copy(stg, out.at[pstage.at[0]], sem_sc).wait()
            fill_dummies()

        def process_group(gi, bank, cursor):
            for b in range(G):
                rel = gi * G + b
                s = lo + rel

                @pl.when((s < hi) & (occ_at(rel) > 0))
                def _(b=b):
                    pltpu.make_async_copy(tab.at[:, pl.ds(0, 128)],
                                          bufs.at[bank * G + b],
                                          sems.at[bank * G + b]).wait()

            def vreg_work(k2, cursor):
                key16 = lkey[pl.ds(16 * k2, 16)]
                rel16 = lax.shift_right_logical(key16, 21)
                scr[pl.ds(0, 16)] = key16
                for b in range(G):
                    m0 = rel16 == (gi * G + b)
                    c16 = plsc.all_reduce_population_count(m0)[0]

                    def match_body(i, carry, b=b):
                        cursor, m = carry
                        lane = plsc.all_reduce_ffs(m)[0]
                        key = scr[pl.ds(lane, 16)][0]
                        lvec = ones16 * (lax.shift_right_logical(key, 14) & 127)
                        p = key & 0x3FFF
                        row = cursor & (FLUSH - 1)
                        for c in range(4):
                            d_vec = jnp.minimum(iota + 16 * c, D - 1)
                            vals = plsc.load_gather(bufs.at[bank * G + b],
                                                    [d_vec, lvec])
                            stg[row, pl.ds(16 * c, 16)] = vals
                        base16 = row - (row & 15)
                        pstage[0, pl.ds(base16, 16)] = jnp.where(
                            iota == (row & 15), p,
                            pstage[0, pl.ds(base16, 16)])

                        @pl.when(row == FLUSH - 1)
                        def _():
                            flush()

                        return cursor + 1, m & (iota != lane)

                    cursor, _ = lax.fori_loop(0, c16, match_body, (cursor, m0))
                return cursor

            return lax.fori_loop(0, nvl, vreg_work, cursor)

        fire_group(0, 0)

        def body2(i, cursor):
            fire_group(2 * i + 1, 1)
            cursor = process_group(2 * i, 0, cursor)
            fire_group(2 * i + 2, 0)
            cursor = process_group(2 * i + 1, 1, cursor)
            return cursor

        lax.fori_loop(0, (NGRP + 1) // 2, body2, 0)
        flush()

    return gk


def _make_tc_proj(B, DP, P, BB):
    grid = B // BB

    def proj_kernel(u_ref, id_ref, ww_ref, bw_ref, wa_ref, ba_ref, ow_ref, oa_ref):
        mask = (id_ref[...] != 0).astype(jnp.float32)
        x = u_ref[...] * mask
        yw = jnp.dot(x, ww_ref[...], preferred_element_type=jnp.float32) + bw_ref[...]
        ya = jnp.dot(x, wa_ref[...], preferred_element_type=jnp.float32) + ba_ref[...]
        ow_ref[...] = jnp.maximum(yw, 0.0)
        oa_ref[...] = jnp.maximum(ya, 0.0)

    return pl.pallas_call(
        proj_kernel,
        grid=(grid,),
        in_specs=[
            pl.BlockSpec((BB, DP), lambda i: (i, 0)),
            pl.BlockSpec((BB, 1), lambda i: (i, 0)),
            pl.BlockSpec((DP, P), lambda i: (0, 0)),
            pl.BlockSpec((1, P), lambda i: (0, 0)),
            pl.BlockSpec((DP, P), lambda i: (0, 0)),
            pl.BlockSpec((1, P), lambda i: (0, 0)),
        ],
        out_specs=[
            pl.BlockSpec((BB, P), lambda i: (i, 0)),
            pl.BlockSpec((BB, P), lambda i: (i, 0)),
        ],
        out_shape=[
            jax.ShapeDtypeStruct((B, P), jnp.float32),
            jax.ShapeDtypeStruct((B, P), jnp.float32),
        ],
    )


def kernel(user_id, table, W_word, b_word, W_article, b_article):
    B = user_id.shape[0]
    V, D = table.shape
    P = W_word.shape[1]
    info = plsc.get_sparse_core_info()
    NC, NS = info.num_cores, info.num_subcores

    idx = user_id.astype(jnp.int32)
    u = _make_sc_gather(V, D, B, NC, NS)(table.T, idx)
    Wwp = jnp.pad(W_word, ((0, _DP - D), (0, 0)))
    Wap = jnp.pad(W_article, ((0, _DP - D), (0, 0)))
    proj = _make_tc_proj(B, _DP, P, BB=2048)
    return tuple(
        proj(u, idx.reshape(B, 1), Wwp, b_word.reshape(1, P),
             Wap, b_article.reshape(1, P))
    )


# speedup vs baseline: 2.2436x; 1.5886x over previous
import functools

import jax
import jax.numpy as jnp
from jax import lax
from jax.experimental import pallas as pl
from jax.experimental.pallas import tpu as pltpu
from jax.experimental.pallas import tpu_sc as plsc

_DP = 128


def _make_sc_gather(V, D, B, NC, NS):
    NW = NC * NS
    NSLAB = (V + 127) // 128
    G = 6
    NBANK = 2
    NV = B // 16
    CAP = B + 32
    SPAN = NSLAB // NW + 1
    NGRP = (SPAN + G - 1) // G
    FLUSH = 64
    mesh = plsc.VectorSubcoreMesh(core_axis_name="c", subcore_axis_name="s")

    @functools.partial(
        pl.kernel,
        mesh=mesh,
        out_type=jax.ShapeDtypeStruct((B + 128, _DP), jnp.float32),
        scratch_types=[
            pltpu.VMEM((B,), jnp.int32),
            pltpu.VMEM((CAP,), jnp.int32),
            pltpu.VMEM((NBANK * G * 56, 128), jnp.float32),
            pltpu.VMEM((FLUSH, _DP), jnp.float32),
            pltpu.VMEM((1, FLUSH), jnp.int32),
            pltpu.VMEM((272,), jnp.int32),
            pltpu.VMEM((32,), jnp.int32),
            pltpu.SemaphoreType.DMA((NBANK * G,)),
            pltpu.SemaphoreType.DMA,
        ],
        compiler_params=pltpu.CompilerParams(
            use_tc_tiling_on_sc=True, needs_layout_passes=False),
    )
    def gk(tab, idx_hbm, out, idx_v, lkey, bufs, stg, pstage, occ, scr,
           sems, sem_sc):
        wid = lax.axis_index("s") * NC + lax.axis_index("c")
        lo = (NSLAB * wid) // NW
        hi = (NSLAB * (wid + 1)) // NW
        iota = lax.broadcasted_iota(jnp.int32, (16,), 0)
        zeros16 = jnp.zeros((16,), jnp.float32)
        ones16 = jnp.full((16,), 1, jnp.int32)

        pltpu.sync_copy(idx_hbm, idx_v)

        for k in range(272 // 16):
            occ[pl.ds(16 * k, 16)] = iota * 0

        def scan_body(k, cnt):
            ids = idx_v[pl.ds(16 * k, 16)]
            rel = lax.shift_right_logical(ids, 7) - lo
            m = (rel >= 0) & (rel < hi - lo)
            relc = jnp.clip(rel, 0, 255)
            key = (relc << 21) | ((ids & 127) << 14) | (iota + 16 * k)
            plsc.store_compressed(lkey.at[pl.ds(cnt, 16)], key, mask=m)
            plsc.store_scatter(occ, [relc], ones16, mask=m)
            return cnt + plsc.all_reduce_population_count(m)[0]

        cnt = lax.fori_loop(0, NV, scan_body, 0)
        lkey[pl.ds(cnt, 16)] = iota * 0 - 1
        nvl = (cnt + 15) >> 4

        def zrow(r, c):
            for c8 in range(_DP // 16):
                stg[r, pl.ds(16 * c8, 16)] = zeros16
            return c

        lax.fori_loop(0, FLUSH, zrow, 0)

        def fill_dummies():
            for k in range(FLUSH // 16):
                pstage[0, pl.ds(16 * k, 16)] = B + iota + 16 * k

        fill_dummies()

        def occ_at(rel):
            return occ[pl.ds(jnp.clip(rel, 0, 255), 16)][0]

        def fire_group(gi, bank):
            for b in range(G):
                rel = gi * G + b
                s = lo + rel

                @pl.when((s < hi) & (occ_at(rel) > 0))
                def _(b=b, s=s, rel=rel):
                    pltpu.async_copy(tab.at[:, pl.ds(s * 128, 128)],
                                     bufs.at[pl.ds((bank * G + b) * 56, D)],
                                     sems.at[bank * G + b])

        def flush():
            pltpu.async_copy(stg, out.at[pstage.at[0]], sem_sc).wait()
            fill_dummies()

        def process_group(gi, bank, cursor):
            for b in range(G):
                rel = gi * G + b
                s = lo + rel

                @pl.when((s < hi) & (occ_at(rel) > 0))
                def _(b=b):
                    pltpu.make_async_copy(tab.at[:, pl.ds(0, 128)],
                                          bufs.at[pl.ds((bank * G + b) * 56, D)],
                                          sems.at[bank * G + b]).wait()

            d_vecs = [jnp.minimum(iota + 16 * c, D - 1) for c in range(4)]

            def vreg_work(k2, cursor):
                key16 = lkey[pl.ds(16 * k2, 16)]
                rel16 = lax.shift_right_logical(key16, 21)
                m0 = (rel16 >= gi * G) & (rel16 < gi * G + G)
                c16 = plsc.all_reduce_population_count(m0)[0]

                def match_body(i, carry):
                    cursor, m = carry
                    lane = plsc.all_reduce_ffs(m)[0]
                    key = scr[pl.ds(lane, 16)][0]
                    rel = lax.shift_right_logical(key, 21)
                    srow = (rel - gi * G + bank * G) * 56
                    lvec = ones16 * (lax.shift_right_logical(key, 14) & 127)
                    p = key & 0x3FFF
                    row = cursor & (FLUSH - 1)
                    for c in range(4):
                        vals = plsc.load_gather(bufs, [srow + d_vecs[c], lvec])
                        stg[row, pl.ds(16 * c, 16)] = vals
                    base16 = row - (row & 15)
                    pstage[0, pl.ds(base16, 16)] = jnp.where(
                        iota == (row & 15), p,
                        pstage[0, pl.ds(base16, 16)])

                    @pl.when(row == FLUSH - 1)
                    def _():
                        flush()

                    return cursor + 1, m & (iota != lane)

                def run(cursor):
                    scr[pl.ds(0, 16)] = key16
                    cur, _ = lax.fori_loop(0, c16, match_body, (cursor, m0))
                    return cur

                return lax.cond(c16 > 0, run, lambda cur: cur, cursor)

            return lax.fori_loop(0, nvl, vreg_work, cursor)

        fire_group(0, 0)

        def body2(i, cursor):
            fire_group(2 * i + 1, 1)
            cursor = process_group(2 * i, 0, cursor)
            fire_group(2 * i + 2, 0)
            cursor = process_group(2 * i + 1, 1, cursor)
            return cursor

        lax.fori_loop(0, (NGRP + 1) // 2, body2, 0)
        flush()

    return gk


def _make_tc_proj(B, DP, P, BB):
    grid = B // BB

    def proj_kernel(u_ref, id_ref, ww_ref, bw_ref, wa_ref, ba_ref, ow_ref, oa_ref):
        mask = (id_ref[...] != 0).astype(jnp.float32)
        x = u_ref[...] * mask
        yw = jnp.dot(x, ww_ref[...], preferred_element_type=jnp.float32) + bw_ref[...]
        ya = jnp.dot(x, wa_ref[...], preferred_element_type=jnp.float32) + ba_ref[...]
        ow_ref[...] = jnp.maximum(yw, 0.0)
        oa_ref[...] = jnp.maximum(ya, 0.0)

    return pl.pallas_call(
        proj_kernel,
        grid=(grid,),
        in_specs=[
            pl.BlockSpec((BB, DP), lambda i: (i, 0)),
            pl.BlockSpec((BB, 1), lambda i: (i, 0)),
            pl.BlockSpec((DP, P), lambda i: (0, 0)),
            pl.BlockSpec((1, P), lambda i: (0, 0)),
            pl.BlockSpec((DP, P), lambda i: (0, 0)),
            pl.BlockSpec((1, P), lambda i: (0, 0)),
        ],
        out_specs=[
            pl.BlockSpec((BB, P), lambda i: (i, 0)),
            pl.BlockSpec((BB, P), lambda i: (i, 0)),
        ],
        out_shape=[
            jax.ShapeDtypeStruct((B, P), jnp.float32),
            jax.ShapeDtypeStruct((B, P), jnp.float32),
        ],
    )


def kernel(user_id, table, W_word, b_word, W_article, b_article):
    B = user_id.shape[0]
    V, D = table.shape
    P = W_word.shape[1]
    info = plsc.get_sparse_core_info()
    NC, NS = info.num_cores, info.num_subcores

    idx = user_id.astype(jnp.int32)
    u = _make_sc_gather(V, D, B, NC, NS)(table.T, idx)
    Wwp = jnp.pad(W_word, ((0, _DP - D), (0, 0)))
    Wap = jnp.pad(W_article, ((0, _DP - D), (0, 0)))
    proj = _make_tc_proj(B, _DP, P, BB=2048)
    return tuple(
        proj(u, idx.reshape(B, 1), Wwp, b_word.reshape(1, P),
             Wap, b_article.reshape(1, P))
    )


# speedup vs baseline: 2.5243x; 1.1251x over previous
import functools

import jax
import jax.numpy as jnp
from jax import lax
from jax.experimental import pallas as pl
from jax.experimental.pallas import tpu as pltpu
from jax.experimental.pallas import tpu_sc as plsc

_DP = 128


def _make_sc_gather(V, D, B, NC, NS):
    NW = NC * NS
    NSLAB = (V + 127) // 128
    G = 6
    NBANK = 2
    NV = B // 16
    CAP = B + 32
    SPAN = NSLAB // NW + 1
    NGRP = (SPAN + G - 1) // G
    FLUSH = 64
    mesh = plsc.VectorSubcoreMesh(core_axis_name="c", subcore_axis_name="s")

    @functools.partial(
        pl.kernel,
        mesh=mesh,
        out_type=jax.ShapeDtypeStruct((B + 128, _DP), jnp.float32),
        scratch_types=[
            pltpu.VMEM((B,), jnp.int32),
            pltpu.VMEM((CAP,), jnp.int32),
            pltpu.VMEM((NBANK * G * 56, 128), jnp.float32),
            pltpu.VMEM((FLUSH, _DP), jnp.float32),
            pltpu.VMEM((1, FLUSH), jnp.int32),
            pltpu.VMEM((272,), jnp.int32),
            pltpu.VMEM((32,), jnp.int32),
            pltpu.SemaphoreType.DMA((NBANK * G,)),
            pltpu.SemaphoreType.DMA,
        ],
        compiler_params=pltpu.CompilerParams(
            use_tc_tiling_on_sc=True, needs_layout_passes=False),
    )
    def gk(tab, idx_hbm, out, idx_v, lkey, bufs, stg, pstage, occ, scr,
           sems, sem_sc):
        wid = lax.axis_index("s") * NC + lax.axis_index("c")
        lo = (NSLAB * wid) // NW
        hi = (NSLAB * (wid + 1)) // NW
        iota = lax.broadcasted_iota(jnp.int32, (16,), 0)
        zeros16 = jnp.zeros((16,), jnp.float32)
        ones16 = jnp.full((16,), 1, jnp.int32)

        pltpu.sync_copy(idx_hbm, idx_v)

        for k in range(272 // 16):
            occ[pl.ds(16 * k, 16)] = iota * 0

        def scan_body(k, cnt):
            ids = idx_v[pl.ds(16 * k, 16)]
            rel = lax.shift_right_logical(ids, 7) - lo
            m = (rel >= 0) & (rel < hi - lo)
            relc = jnp.clip(rel, 0, 255)
            key = (relc << 21) | ((ids & 127) << 14) | (iota + 16 * k)
            plsc.store_compressed(lkey.at[pl.ds(cnt, 16)], key, mask=m)
            plsc.store_scatter(occ, [relc], ones16, mask=m)
            return cnt + plsc.all_reduce_population_count(m)[0]

        cnt = lax.fori_loop(0, NV, scan_body, 0)
        lkey[pl.ds(cnt, 16)] = iota * 0 - 1
        nvl = (cnt + 15) >> 4

        def zrow(r, c):
            for c8 in range(_DP // 16):
                stg[r, pl.ds(16 * c8, 16)] = zeros16
            return c

        lax.fori_loop(0, FLUSH, zrow, 0)

        def fill_dummies():
            for k in range(FLUSH // 16):
                pstage[0, pl.ds(16 * k, 16)] = B + iota + 16 * k

        fill_dummies()

        def occ_at(rel):
            return occ[pl.ds(jnp.clip(rel, 0, 255), 16)][0]

        def fire_group(gi, bank):
            for b in range(G):
                rel = gi * G + b
                s = lo + rel

                @pl.when((s < hi) & (occ_at(rel) > 0))
                def _(b=b, s=s, rel=rel):
                    pltpu.async_copy(tab.at[:, pl.ds(s * 128, 128)],
                                     bufs.at[pl.ds((bank * G + b) * 56, D)],
                                     sems.at[bank * G + b])

        def flush():
            pltpu.async_copy(stg, out.at[pstage.at[0]], sem_sc).wait()
            fill_dummies()

        def process_group(gi, bank, cursor):
            for b in range(G):
                rel = gi * G + b
                s = lo + rel

                @pl.when((s < hi) & (occ_at(rel) > 0))
                def _(b=b):
                    pltpu.make_async_copy(tab.at[:, pl.ds(0, 128)],
                                          bufs.at[pl.ds((bank * G + b) * 56, D)],
                                          sems.at[bank * G + b]).wait()

            d_vecs = [jnp.minimum(iota + 16 * c, D - 1) for c in range(4)]

            def vreg_work(k2, cursor):
                key16 = lkey[pl.ds(16 * k2, 16)]
                rel16 = lax.shift_right_logical(key16, 21)
                m0 = (rel16 >= gi * G) & (rel16 < gi * G + G)
                c16 = plsc.all_reduce_population_count(m0)[0]

                def match_body(i, carry):
                    cursor, m = carry
                    lane = plsc.all_reduce_ffs(m)[0]
                    key = scr[pl.ds(lane, 16)][0]
                    rel = lax.shift_right_logical(key, 21)
                    srow = (rel - gi * G + bank * G) * 56
                    lvec = ones16 * (lax.shift_right_logical(key, 14) & 127)
                    p = key & 0x3FFF
                    row = cursor & (FLUSH - 1)
                    for c in range(4):
                        vals = plsc.load_gather(bufs, [srow + d_vecs[c], lvec])
                        stg[row, pl.ds(16 * c, 16)] = vals
                    base16 = row - (row & 15)
                    pstage[0, pl.ds(base16, 16)] = jnp.where(
                        iota == (row & 15), p,
                        pstage[0, pl.ds(base16, 16)])

                    @pl.when(row == FLUSH - 1)
                    def _():
                        flush()

                    return cursor + 1, m & (iota != lane)

                def run(cursor):
                    scr[pl.ds(0, 16)] = key16
                    cur, _ = lax.fori_loop(0, c16, match_body, (cursor, m0))
                    return cur

                return lax.cond(c16 > 0, run, lambda cur: cur, cursor)

            return lax.fori_loop(0, nvl, vreg_work, cursor)

        fire_group(0, 0)

        def body2(i, cursor):
            fire_group(2 * i + 1, 1)
            cursor = process_group(2 * i, 0, cursor)
            fire_group(2 * i + 2, 0)
            cursor = process_group(2 * i + 1, 1, cursor)
            return cursor

        lax.fori_loop(0, (NGRP + 1) // 2, body2, 0)
        flush()

    return gk


def _make_tc_proj(B, DP, P, BB):
    grid = B // BB

    def proj_kernel(u_ref, id_ref, ww_ref, bw_ref, wa_ref, ba_ref, ow_ref, oa_ref):
        mask = (id_ref[...] != 0).astype(jnp.float32)
        x = u_ref[...] * mask
        dn = (((0,), (0,)), ((), ()))
        yw = lax.dot_general(ww_ref[...], x, dn,
                             preferred_element_type=jnp.float32) + bw_ref[...]
        ya = lax.dot_general(wa_ref[...], x, dn,
                             preferred_element_type=jnp.float32) + ba_ref[...]
        ow_ref[...] = jnp.maximum(yw, 0.0)
        oa_ref[...] = jnp.maximum(ya, 0.0)

    return pl.pallas_call(
        proj_kernel,
        grid=(grid,),
        in_specs=[
            pl.BlockSpec((DP, BB), lambda i: (0, i)),
            pl.BlockSpec((1, BB), lambda i: (0, i)),
            pl.BlockSpec((DP, P), lambda i: (0, 0)),
            pl.BlockSpec((P, 1), lambda i: (0, 0)),
            pl.BlockSpec((DP, P), lambda i: (0, 0)),
            pl.BlockSpec((P, 1), lambda i: (0, 0)),
        ],
        out_specs=[
            pl.BlockSpec((P, BB), lambda i: (0, i)),
            pl.BlockSpec((P, BB), lambda i: (0, i)),
        ],
        out_shape=[
            jax.ShapeDtypeStruct((P, B), jnp.float32),
            jax.ShapeDtypeStruct((P, B), jnp.float32),
        ],
    )


def kernel(user_id, table, W_word, b_word, W_article, b_article):
    B = user_id.shape[0]
    V, D = table.shape
    P = W_word.shape[1]
    info = plsc.get_sparse_core_info()
    NC, NS = info.num_cores, info.num_subcores

    idx = user_id.astype(jnp.int32)
    u = _make_sc_gather(V, D, B, NC, NS)(table.T, idx)
    uT = u[:B].T
    Wwp = jnp.pad(W_word, ((0, _DP - D), (0, 0)))
    Wap = jnp.pad(W_article, ((0, _DP - D), (0, 0)))
    proj = _make_tc_proj(B, _DP, P, BB=2048)
    ywT, yaT = proj(uT, idx.reshape(1, B), Wwp, b_word.reshape(P, 1),
                    Wap, b_article.reshape(P, 1))
    return (ywT.T, yaT.T)


# speedup vs baseline: 2.5479x; 1.0094x over previous
import functools

import jax
import jax.numpy as jnp
from jax import lax
from jax.experimental import pallas as pl
from jax.experimental.pallas import tpu as pltpu
from jax.experimental.pallas import tpu_sc as plsc

_DP = 128


def _make_sc_gather(V, D, B, NC, NS):
    NW = NC * NS
    NSLAB = (V + 127) // 128
    G = 6
    NBANK = 2
    NV = B // 16
    CAP = B + 32
    SPAN = NSLAB // NW + 1
    NGRP = (SPAN + G - 1) // G
    FLUSH = 64
    mesh = plsc.VectorSubcoreMesh(core_axis_name="c", subcore_axis_name="s")

    @functools.partial(
        pl.kernel,
        mesh=mesh,
        out_type=jax.ShapeDtypeStruct((B + 128, _DP), jnp.float32),
        scratch_types=[
            pltpu.VMEM((B,), jnp.int32),
            pltpu.VMEM((CAP,), jnp.int32),
            pltpu.VMEM((NBANK * G * 56, 128), jnp.float32),
            pltpu.VMEM((FLUSH, _DP), jnp.float32),
            pltpu.VMEM((1, FLUSH), jnp.int32),
            pltpu.VMEM((272,), jnp.int32),
            pltpu.VMEM((32,), jnp.int32),
            pltpu.SemaphoreType.DMA((NBANK * G,)),
            pltpu.SemaphoreType.DMA,
        ],
        compiler_params=pltpu.CompilerParams(
            use_tc_tiling_on_sc=True, needs_layout_passes=False),
    )
    def gk(tab, idx_hbm, out, idx_v, lkey, bufs, stg, pstage, occ, scr,
           sems, sem_sc):
        wid = lax.axis_index("s") * NC + lax.axis_index("c")
        lo = (NSLAB * wid) // NW
        hi = (NSLAB * (wid + 1)) // NW
        iota = lax.broadcasted_iota(jnp.int32, (16,), 0)
        zeros16 = jnp.zeros((16,), jnp.float32)
        ones16 = jnp.full((16,), 1, jnp.int32)

        pltpu.sync_copy(idx_hbm, idx_v)

        for k in range(272 // 16):
            occ[pl.ds(16 * k, 16)] = iota * 0

        def scan_body(k, cnt):
            ids = idx_v[pl.ds(16 * k, 16)]
            rel = lax.shift_right_logical(ids, 7) - lo
            m = (rel >= 0) & (rel < hi - lo)
            relc = jnp.clip(rel, 0, 255)
            key = (relc << 21) | ((ids & 127) << 14) | (iota + 16 * k)
            plsc.store_compressed(lkey.at[pl.ds(cnt, 16)], key, mask=m)
            plsc.store_scatter(occ, [relc], ones16, mask=m)
            return cnt + plsc.all_reduce_population_count(m)[0]

        cnt = lax.fori_loop(0, NV, scan_body, 0)
        lkey[pl.ds(cnt, 16)] = iota * 0 - 1
        nvl = (cnt + 15) >> 4

        def zrow(r, c):
            for c8 in range(_DP // 16):
                stg[r, pl.ds(16 * c8, 16)] = zeros16
            return c

        lax.fori_loop(0, FLUSH, zrow, 0)

        def fill_dummies():
            for k in range(FLUSH // 16):
                pstage[0, pl.ds(16 * k, 16)] = B + iota + 16 * k

        fill_dummies()

        def occ_at(rel):
            return occ[pl.ds(jnp.clip(rel, 0, 255), 16)][0]

        def fire_group(gi, bank):
            for b in range(G):
                rel = gi * G + b
                s = lo + rel

                @pl.when((s < hi) & (occ_at(rel) > 0))
                def _(b=b, s=s, rel=rel):
                    pltpu.async_copy(tab.at[:, pl.ds(s * 128, 128)],
                                     bufs.at[pl.ds((bank * G + b) * 56, D)],
                                     sems.at[bank * G + b])

        def flush():
            pltpu.async_copy(stg, out.at[pstage.at[0]], sem_sc).wait()
            fill_dummies()

        def process_group(gi, bank, cursor):
            for b in range(G):
                rel = gi * G + b
                s = lo + rel

                @pl.when((s < hi) & (occ_at(rel) > 0))
                def _(b=b):
                    pltpu.make_async_copy(tab.at[:, pl.ds(0, 128)],
                                          bufs.at[pl.ds((bank * G + b) * 56, D)],
                                          sems.at[bank * G + b]).wait()

            d_vecs = [jnp.minimum(iota + 16 * c, D - 1) for c in range(4)]

            def vreg_work(k2, cursor):
                key16 = lkey[pl.ds(16 * k2, 16)]
                rel16 = lax.shift_right_logical(key16, 21)
                m0 = (rel16 >= gi * G) & (rel16 < gi * G + G)
                c16 = plsc.all_reduce_population_count(m0)[0]

                def match_body(i, carry):
                    cursor, m = carry
                    lane = plsc.all_reduce_ffs(m)[0]
                    key = scr[pl.ds(lane, 16)][0]
                    rel = lax.shift_right_logical(key, 21)
                    srow = (rel - gi * G + bank * G) * 56
                    lvec = ones16 * (lax.shift_right_logical(key, 14) & 127)
                    p = key & 0x3FFF
                    row = cursor & (FLUSH - 1)
                    for c in range(4):
                        vals = plsc.load_gather(bufs, [srow + d_vecs[c], lvec])
                        stg[row, pl.ds(16 * c, 16)] = vals
                    base16 = row - (row & 15)
                    pstage[0, pl.ds(base16, 16)] = jnp.where(
                        iota == (row & 15), p,
                        pstage[0, pl.ds(base16, 16)])

                    @pl.when(row == FLUSH - 1)
                    def _():
                        flush()

                    return cursor + 1, m & (iota != lane)

                def run(cursor):
                    scr[pl.ds(0, 16)] = key16
                    cur, _ = lax.fori_loop(0, c16, match_body, (cursor, m0))
                    return cur

                return lax.cond(c16 > 0, run, lambda cur: cur, cursor)

            return lax.fori_loop(0, nvl, vreg_work, cursor)

        fire_group(0, 0)

        def body2(i, cursor):
            fire_group(2 * i + 1, 1)
            cursor = process_group(2 * i, 0, cursor)
            fire_group(2 * i + 2, 0)
            cursor = process_group(2 * i + 1, 1, cursor)
            return cursor

        lax.fori_loop(0, (NGRP + 1) // 2, body2, 0)
        flush()

    return gk


def _make_tc_proj(B, DP, P, BB):
    grid = B // BB

    def proj_kernel(u_ref, id_ref, ww_ref, bw_ref, wa_ref, ba_ref, ow_ref, oa_ref):
        mask = (id_ref[...] != 0).astype(jnp.float32)
        x = u_ref[...] * mask
        dn = (((0,), (0,)), ((), ()))
        yw = lax.dot_general(ww_ref[...], x, dn,
                             preferred_element_type=jnp.float32) + bw_ref[...]
        ya = lax.dot_general(wa_ref[...], x, dn,
                             preferred_element_type=jnp.float32) + ba_ref[...]
        ow_ref[...] = jnp.maximum(yw, 0.0)
        oa_ref[...] = jnp.maximum(ya, 0.0)

    return pl.pallas_call(
        proj_kernel,
        grid=(grid,),
        in_specs=[
            pl.BlockSpec((DP, BB), lambda i: (0, i)),
            pl.BlockSpec((1, BB), lambda i: (0, i)),
            pl.BlockSpec((DP, P), lambda i: (0, 0)),
            pl.BlockSpec((P, 1), lambda i: (0, 0)),
            pl.BlockSpec((DP, P), lambda i: (0, 0)),
            pl.BlockSpec((P, 1), lambda i: (0, 0)),
        ],
        out_specs=[
            pl.BlockSpec((P, BB), lambda i: (0, i)),
            pl.BlockSpec((P, BB), lambda i: (0, i)),
        ],
        out_shape=[
            jax.ShapeDtypeStruct((P, B), jnp.float32),
            jax.ShapeDtypeStruct((P, B), jnp.float32),
        ],
    )


def kernel(user_id, table, W_word, b_word, W_article, b_article):
    B = user_id.shape[0]
    V, D = table.shape
    P = W_word.shape[1]
    info = plsc.get_sparse_core_info()
    NC, NS = info.num_cores, info.num_subcores

    idx = user_id.astype(jnp.int32)
    u = _make_sc_gather(V, D, B, NC, NS)(table.T, idx)
    uT = u[:B].T
    Wwp = jnp.pad(W_word, ((0, _DP - D), (0, 0)))
    Wap = jnp.pad(W_article, ((0, _DP - D), (0, 0)))
    proj = _make_tc_proj(B, _DP, P, BB=4096)
    ywT, yaT = proj(uT, idx.reshape(1, B), Wwp, b_word.reshape(P, 1),
                    Wap, b_article.reshape(P, 1))
    return (ywT.T, yaT.T)


# speedup vs baseline: 2.6416x; 1.0368x over previous
import functools

import jax
import jax.numpy as jnp
from jax import lax
from jax.experimental import pallas as pl
from jax.experimental.pallas import tpu as pltpu
from jax.experimental.pallas import tpu_sc as plsc

_DP = 128


def _make_sc_gather(V, D, B, NC, NS):
    NW = NC * NS
    NSLAB = (V + 127) // 128
    G = 6
    NBANK = 2
    NV = B // 16
    CAP = B + 32
    SPAN = NSLAB // NW + 1
    NGRP = (SPAN + G - 1) // G
    FLUSH = 64
    mesh = plsc.VectorSubcoreMesh(core_axis_name="c", subcore_axis_name="s")

    @functools.partial(
        pl.kernel,
        mesh=mesh,
        out_type=jax.ShapeDtypeStruct((B + 128, _DP), jnp.float32),
        scratch_types=[
            pltpu.VMEM((B,), jnp.int32),
            pltpu.VMEM((CAP,), jnp.int32),
            pltpu.VMEM((NBANK * G * 56, 128), jnp.float32),
            pltpu.VMEM((FLUSH, _DP), jnp.float32),
            pltpu.VMEM((1, FLUSH), jnp.int32),
            pltpu.VMEM((272,), jnp.int32),
            pltpu.VMEM((32,), jnp.int32),
            pltpu.SemaphoreType.DMA((NBANK * G,)),
            pltpu.SemaphoreType.DMA,
        ],
        compiler_params=pltpu.CompilerParams(
            use_tc_tiling_on_sc=True, needs_layout_passes=False),
    )
    def gk(tab, idx_hbm, out, idx_v, lkey, bufs, stg, pstage, occ, scr,
           sems, sem_sc):
        wid = lax.axis_index("s") * NC + lax.axis_index("c")
        lo = (NSLAB * wid) // NW
        hi = (NSLAB * (wid + 1)) // NW
        iota = lax.broadcasted_iota(jnp.int32, (16,), 0)
        zeros16 = jnp.zeros((16,), jnp.float32)
        ones16 = jnp.full((16,), 1, jnp.int32)

        pltpu.sync_copy(idx_hbm, idx_v)

        for k in range(272 // 16):
            occ[pl.ds(16 * k, 16)] = iota * 0

        def scan_body(k, cnt):
            ids = idx_v[pl.ds(16 * k, 16)]
            rel = lax.shift_right_logical(ids, 7) - lo
            m = (rel >= 0) & (rel < hi - lo)
            relc = jnp.clip(rel, 0, 255)
            key = (relc << 21) | ((ids & 127) << 14) | (iota + 16 * k)
            plsc.store_compressed(lkey.at[pl.ds(cnt, 16)], key, mask=m)
            plsc.store_scatter(occ, [relc], ones16, mask=m)
            return cnt + plsc.all_reduce_population_count(m)[0]

        cnt = lax.fori_loop(0, NV, scan_body, 0)
        lkey[pl.ds(cnt, 16)] = iota * 0 - 1
        nvl = (cnt + 15) >> 4

        def zrow(r, c):
            for c8 in range(_DP // 16):
                stg[r, pl.ds(16 * c8, 16)] = zeros16
            return c

        lax.fori_loop(0, FLUSH, zrow, 0)

        def fill_dummies():
            for k in range(FLUSH // 16):
                pstage[0, pl.ds(16 * k, 16)] = B + iota + 16 * k

        fill_dummies()

        def occ_at(rel):
            return occ[pl.ds(jnp.clip(rel, 0, 255), 16)][0]

        def fire_group(gi, bank):
            for b in range(G):
                rel = gi * G + b
                s = lo + rel

                @pl.when((s < hi) & (occ_at(rel) > 0))
                def _(b=b, s=s, rel=rel):
                    pltpu.async_copy(tab.at[:, pl.ds(s * 128, 128)],
                                     bufs.at[pl.ds((bank * G + b) * 56, D)],
                                     sems.at[bank * G + b])

        def flush():
            pltpu.async_copy(stg, out.at[pstage.at[0]], sem_sc).wait()
            fill_dummies()

        def process_group(gi, bank, cursor):
            for b in range(G):
                rel = gi * G + b
                s = lo + rel

                @pl.when((s < hi) & (occ_at(rel) > 0))
                def _(b=b):
                    pltpu.make_async_copy(tab.at[:, pl.ds(0, 128)],
                                          bufs.at[pl.ds((bank * G + b) * 56, D)],
                                          sems.at[bank * G + b]).wait()

            d_vecs = [jnp.minimum(iota + 16 * c, D - 1) for c in range(4)]

            def vreg_work(k2, cursor):
                key16 = lkey[pl.ds(16 * k2, 16)]
                rel16 = lax.shift_right_logical(key16, 21)
                m0 = (rel16 >= gi * G) & (rel16 < gi * G + G)
                c16 = plsc.all_reduce_population_count(m0)[0]

                def match_body(i, carry):
                    cursor, m = carry
                    lane = plsc.all_reduce_ffs(m)[0]
                    key = scr[pl.ds(lane, 16)][0]
                    rel = lax.shift_right_logical(key, 21)
                    srow = (rel - gi * G + bank * G) * 56
                    lvec = ones16 * (lax.shift_right_logical(key, 14) & 127)
                    p = key & 0x3FFF
                    row = cursor & (FLUSH - 1)
                    for c in range(4):
                        vals = plsc.load_gather(bufs, [srow + d_vecs[c], lvec])
                        stg[row, pl.ds(16 * c, 16)] = vals
                    base16 = row - (row & 15)
                    pstage[0, pl.ds(base16, 16)] = jnp.where(
                        iota == (row & 15), p,
                        pstage[0, pl.ds(base16, 16)])

                    @pl.when(row == FLUSH - 1)
                    def _():
                        flush()

                    return cursor + 1, m & (iota != lane)

                def run(cursor):
                    scr[pl.ds(0, 16)] = key16
                    cur, _ = lax.fori_loop(0, c16, match_body, (cursor, m0))
                    return cur

                return lax.cond(c16 > 0, run, lambda cur: cur, cursor)

            return lax.fori_loop(0, nvl, vreg_work, cursor)

        fire_group(0, 0)

        def body2(i, cursor):
            fire_group(2 * i + 1, 1)
            cursor = process_group(2 * i, 0, cursor)
            fire_group(2 * i + 2, 0)
            cursor = process_group(2 * i + 1, 1, cursor)
            return cursor

        lax.fori_loop(0, (NGRP + 1) // 2, body2, 0)
        flush()

    return gk


def _make_tc_proj(B, DP, P, BB):
    grid = B // BB

    def proj_kernel(u_ref, id_ref, ww_ref, bw_ref, wa_ref, ba_ref, ow_ref, oa_ref):
        mask = (id_ref[...] != 0).astype(jnp.float32)
        x = u_ref[...] * mask
        dn = (((0,), (0,)), ((), ()))
        yw = lax.dot_general(ww_ref[...], x, dn,
                             preferred_element_type=jnp.float32) + bw_ref[...]
        ya = lax.dot_general(wa_ref[...], x, dn,
                             preferred_element_type=jnp.float32) + ba_ref[...]
        ow_ref[...] = jnp.maximum(yw, 0.0)
        oa_ref[...] = jnp.maximum(ya, 0.0)

    return pl.pallas_call(
        proj_kernel,
        grid=(grid,),
        in_specs=[
            pl.BlockSpec((DP, BB), lambda i: (0, i)),
            pl.BlockSpec((1, BB), lambda i: (0, i)),
            pl.BlockSpec((DP, P), lambda i: (0, 0)),
            pl.BlockSpec((P, 1), lambda i: (0, 0)),
            pl.BlockSpec((DP, P), lambda i: (0, 0)),
            pl.BlockSpec((P, 1), lambda i: (0, 0)),
        ],
        out_specs=[
            pl.BlockSpec((P, BB), lambda i: (0, i)),
            pl.BlockSpec((P, BB), lambda i: (0, i)),
        ],
        out_shape=[
            jax.ShapeDtypeStruct((P, B), jnp.float32),
            jax.ShapeDtypeStruct((P, B), jnp.float32),
        ],
    )


def kernel(user_id, table, W_word, b_word, W_article, b_article):
    B = user_id.shape[0]
    V, D = table.shape
    P = W_word.shape[1]
    info = plsc.get_sparse_core_info()
    NC, NS = info.num_cores, info.num_subcores

    idx = user_id.astype(jnp.int32)
    u = _make_sc_gather(V, D, B, NC, NS)(table.T, idx)
    uT = u.T
    Wwp = jnp.pad(W_word, ((0, _DP - D), (0, 0)))
    Wap = jnp.pad(W_article, ((0, _DP - D), (0, 0)))
    proj = _make_tc_proj(B, _DP, P, BB=4096)
    ywT, yaT = proj(uT, idx.reshape(1, B), Wwp, b_word.reshape(P, 1),
                    Wap, b_article.reshape(P, 1))
    return (ywT.T, yaT.T)
